# Initial kernel scaffold; baseline (speedup 1.0000x reference)
import jax
import jax.numpy as jnp
from jax.experimental import pallas as pl


def kernel(inp_s, inp_t, edge_index_s, edge_index_t, adj_vals_s, adj_vals_t, W):
    raise NotImplementedError("write your pallas kernel here")


# speedup vs baseline: 3.4310x; 3.4310x over previous
import functools

import jax
import jax.numpy as jnp
from jax import lax
from jax.experimental import pallas as pl
from jax.experimental.pallas import tpu as pltpu
from jax.experimental.pallas import tpu_sc as plsc

N = 10000
E = 320000
D = 128

NC = 2
NS = 16
SUP = 256
SUPS_PER_TILE = 80
EP = NS * SUPS_PER_TILE * SUP
NCHUNK = EP // SUP
WB = 80
NWB = N // WB

MM_BLOCK = 1000


def _mm_body(x_ref, w_ref, o_ref):
    o_ref[...] = jnp.dot(x_ref[...], w_ref[...],
                         preferred_element_type=jnp.float32)


def _project(x2, W):
    grid = (2 * N) // MM_BLOCK
    return pl.pallas_call(
        _mm_body,
        grid=(grid,),
        in_specs=[
            pl.BlockSpec((MM_BLOCK, D), lambda i: (i, 0)),
            pl.BlockSpec((D, D), lambda i: (0, 0)),
        ],
        out_specs=pl.BlockSpec((MM_BLOCK, D), lambda i: (i, 0)),
        out_shape=jax.ShapeDtypeStruct((2 * N, D), jnp.float32),
    )(x2, W)


def _sc_body(h_hbm, src_hbm, dst_hbm, vals_hbm, out_hbm,
             acc, src_v, dst_v, vals_v, rows_v, sem):
    g = lax.axis_index("c")
    s = lax.axis_index("s")

    zero = jnp.zeros((16,), jnp.float32)

    @pl.loop(0, WB)
    def _zero(i):
        for j in range(D // 16):
            rows_v[i, pl.ds(j * 16, 16)] = zero

    nwb = jnp.where(s < NWB - (NWB // NS) * NS, NWB // NS + 1, NWB // NS)

    @pl.loop(0, nwb)
    def _zacc(t):
        pltpu.sync_copy(rows_v.at[pl.ds(0, WB)],
                        acc.at[pl.ds((s + t * NS) * WB, WB)])

    plsc.subcore_barrier()

    @pl.loop(0, SUPS_PER_TILE)
    def _chunk(k):
        chunk_id = s * SUPS_PER_TILE + k
        base_e = g * EP + chunk_id * SUP
        pltpu.sync_copy(src_hbm.at[g * NCHUNK + chunk_id], src_v)
        pltpu.sync_copy(dst_hbm.at[g * NCHUNK + chunk_id], dst_v)
        pltpu.sync_copy(vals_hbm.at[pl.ds(base_e, SUP)], vals_v)

        cps = [pltpu.async_copy(h_hbm.at[src_v.at[q]],
                                rows_v.at[pl.ds(q * 128, 128)], sem)
               for q in range(SUP // 128)]
        for cp in cps:
            cp.wait()

        @pl.loop(0, SUP // 16)
        def _scale(b):
            vv16 = vals_v[pl.ds(b * 16, 16)]
            for e in range(16):
                i = b * 16 + e
                vv = vv16[e]
                for j in range(D // 16):
                    sl = pl.ds(j * 16, 16)
                    rows_v[i, sl] = rows_v[i, sl] * vv

        for q in range(SUP // 128):
            pltpu.sync_copy(rows_v.at[pl.ds(q * 128, 128)],
                            acc.at[dst_v.at[q]], add=True)

    plsc.subcore_barrier()

    @pl.loop(0, nwb)
    def _wb(t):
        rb = (s + t * NS) * WB
        pltpu.sync_copy(acc.at[pl.ds(rb, WB)], rows_v.at[pl.ds(0, WB)])

        @pl.loop(0, WB)
        def _relu(i):
            for j in range(D // 16):
                sl = pl.ds(j * 16, 16)
                rows_v[i, sl] = jnp.maximum(rows_v[i, sl], 0.0)

        pltpu.sync_copy(rows_v.at[pl.ds(0, WB)],
                        out_hbm.at[pl.ds(g * N + rb, WB)])


_sc_agg = functools.partial(
    pl.kernel,
    out_type=jax.ShapeDtypeStruct((2 * N, D), jnp.float32),
    mesh=plsc.VectorSubcoreMesh(core_axis_name="c", subcore_axis_name="s",
                                num_cores=NC, num_subcores=NS),
    scratch_types=[
        pltpu.VMEM_SHARED((N, D), jnp.float32),
        pltpu.VMEM((SUP // 128, 128), jnp.int32),
        pltpu.VMEM((SUP // 128, 128), jnp.int32),
        pltpu.VMEM((SUP,), jnp.float32),
        pltpu.VMEM((SUP, D), jnp.float32),
        pltpu.SemaphoreType.DMA,
    ],
)(_sc_body)


def _pad_idx(a, pad_val):
    return jnp.concatenate(
        [a, jnp.full((EP - E,), pad_val, dtype=jnp.int32)])


def kernel(inp_s, inp_t, edge_index_s, edge_index_t,
           adj_vals_s, adj_vals_t, W):
    x2 = jnp.concatenate([inp_s, inp_t], axis=0)
    h2 = _project(x2, W)

    src2 = jnp.concatenate([
        _pad_idx(edge_index_s[0], 0),
        _pad_idx(edge_index_t[0], 0) + N,
    ]).reshape(2 * NCHUNK, SUP // 128, 128)
    dst2 = jnp.concatenate([
        _pad_idx(edge_index_s[1], 0),
        _pad_idx(edge_index_t[1], 0),
    ]).reshape(2 * NCHUNK, SUP // 128, 128)
    zpad = jnp.zeros((EP - E,), jnp.float32)
    vals2 = jnp.concatenate([adj_vals_s, zpad, adj_vals_t, zpad])

    out2 = _sc_agg(h2, src2, dst2, vals2)
    return (out2[:N], out2[N:])
